# baseline (speedup 1.0000x reference)
import functools

import jax
import jax.numpy as jnp
from jax import lax
from jax.experimental import pallas as pl
from jax.experimental.pallas import tpu as pltpu
from jax.experimental.pallas import tpu_sc as plsc

F = 26
V = 100000
D = 32
B = 4096
NC = 2
NS = 16
NW = NC * NS
BPW = B // NW
L = 16


def _sc_body(xcat_t_hbm, tab_hbm, out_hbm, idx_v, acc_v, sem0, sem):
    c = lax.axis_index("c")
    s = lax.axis_index("s")
    wid = s * NC + c
    base = wid * BPW

    pltpu.sync_copy(xcat_t_hbm.at[:, pl.ds(base, BPW)], idx_v)

    pltpu.async_copy(tab_hbm.at[0].at[idx_v.at[0]], acc_v, sem0).wait()

    for j in range(1, F):
        pltpu.async_copy(tab_hbm.at[j].at[idx_v.at[j]], acc_v, sem, add=True)
    for j in range(1, F):
        pltpu.make_async_copy(tab_hbm.at[j].at[idx_v.at[j]], acc_v, sem).wait()

    pltpu.sync_copy(acc_v, out_hbm.at[pl.ds(base, BPW), :])


@jax.jit
def kernel(x_cat, tables):
    xcat_t = x_cat.T
    mesh = plsc.VectorSubcoreMesh(core_axis_name="c", subcore_axis_name="s")
    run = pl.kernel(
        _sc_body,
        out_type=jax.ShapeDtypeStruct((B, D), jnp.float32),
        mesh=mesh,
        scratch_types=[
            pltpu.VMEM((F, BPW), jnp.int32),
            pltpu.VMEM((BPW, D), jnp.float32),
            pltpu.SemaphoreType.DMA,
            pltpu.SemaphoreType.DMA,
        ],
        compiler_params=pltpu.CompilerParams(use_tc_tiling_on_sc=False),
    )
    return run(xcat_t, tables)
